# baseline (speedup 1.0000x reference)
import jax
import jax.numpy as jnp
from jax.experimental import pallas as pl
from jax.experimental.pallas import tpu as pltpu
from jax.experimental.pallas import tpu_sc as plsc

_BR = 16
_LANES = 16


def _sc_body(x_vmem, pe_vmem, o_vmem):
    ncols = x_vmem.shape[1]

    @pl.loop(0, _BR)
    def _(r):
        @pl.loop(0, ncols, step=_LANES)
        def _(c):
            slc = (pl.ds(r, 1), pl.ds(c, _LANES))
            o_vmem.at[*slc][...] = x_vmem.at[*slc][...] + pe_vmem.at[*slc][...]


def kernel(x, pe_weight):
    B, S, D = x.shape
    xf = x.reshape(B * S, D)
    n_pe_blocks = S // _BR

    @pl.kernel(
        out_type=jax.ShapeDtypeStruct((B * S, D), x.dtype),
        mesh=plsc.VectorSubcoreMesh(core_axis_name="c", subcore_axis_name="s"),
    )
    def run(x_hbm, pe_hbm, o_hbm):
        pltpu.emit_pipeline(
            _sc_body,
            grid=(B * S // _BR,),
            in_specs=[
                pl.BlockSpec((_BR, D), lambda i: (i, 0)),
                pl.BlockSpec((_BR, D), lambda i: (i % n_pe_blocks, 0)),
            ],
            out_specs=[pl.BlockSpec((_BR, D), lambda i: (i, 0))],
            core_axis_name=("c", "s"),
            dimension_semantics=(pltpu.PARALLEL,),
        )(x_hbm, pe_hbm, o_hbm)

    return run(xf, pe_weight).reshape(B, S, D)
